# baseline (speedup 1.0000x reference)
import jax
import jax.numpy as jnp
from jax.experimental import pallas as pl

_O = 96
_I = 96
_K = 864
_H = 56
_W = 56
_P = _H * _W
_PAD = 64
_HALF = 3.0
_BOUND = 1.5


def _quant_body(w_ref, qw_ref):
    w = w_ref[...]
    step = jnp.max(jnp.abs(w)) / _HALF
    ws = w / step
    col = jax.lax.broadcasted_iota(jnp.int32, (_O, _K), 1)
    g = col & 3
    sum4 = ws + jnp.roll(ws, -1, 1) + jnp.roll(ws, -2, 1) + jnp.roll(ws, -3, 1)
    base = jnp.where(g == 0, sum4, 0.0)
    mean = (base + jnp.roll(base, 1, 1) + jnp.roll(base, 2, 1)
            + jnp.roll(base, 3, 1)) * 0.25
    cent = jnp.round(jnp.clip(mean, -_BOUND, _BOUND))
    dev = jnp.round(jnp.clip(ws - cent, -_BOUND, _BOUND))
    qw_ref[...] = (dev + cent) * step


def _conv_body(mask_ref, xp_ref, wt_ref, bias_ref, out_ref):
    xp = xp_ref[0]
    mL = mask_ref[0:1, :]
    mR = mask_ref[1:2, :]
    acc = jnp.zeros((_O, _P), jnp.float32)
    for t in range(9):
        dh, dw = t // 3 - 1, t % 3 - 1
        s = dh * _W + dw
        xs = xp[:, _PAD + s:_PAD + s + _P]
        if dw == -1:
            xs = xs * mL
        elif dw == 1:
            xs = xs * mR
        acc = acc + jnp.dot(wt_ref[t], xs, preferred_element_type=jnp.float32)
    out_ref[0] = acc + bias_ref[...]


def kernel(x, weight, bias):
    n = x.shape[0]
    wf = weight.reshape(_O, _K)

    qw = pl.pallas_call(
        _quant_body,
        out_shape=jax.ShapeDtypeStruct((_O, _K), jnp.float32),
    )(wf)

    wt = qw.reshape(_O, _I, 9).transpose(2, 0, 1)

    xf = x.reshape(n, _I, _P)
    xp = jnp.pad(xf, ((0, 0), (0, 0), (_PAD, _PAD)))
    colp = jnp.arange(_P) % _W
    masks = jnp.stack([(colp >= 1).astype(jnp.float32),
                       (colp <= _W - 2).astype(jnp.float32)])

    out = pl.pallas_call(
        _conv_body,
        grid=(n,),
        in_specs=[
            pl.BlockSpec((2, _P), lambda i: (0, 0)),
            pl.BlockSpec((1, _I, _P + 2 * _PAD), lambda i: (i, 0, 0)),
            pl.BlockSpec((9, _O, _I), lambda i: (0, 0, 0)),
            pl.BlockSpec((_O, 1), lambda i: (0, 0)),
        ],
        out_specs=pl.BlockSpec((1, _O, _P), lambda i: (i, 0, 0)),
        out_shape=jax.ShapeDtypeStruct((n, _O, _P), jnp.float32),
    )(masks, xp, wt, bias.reshape(_O, 1))

    return out.reshape(n, _O, _H, _W)
